# Initial kernel scaffold; baseline (speedup 1.0000x reference)
import jax
import jax.numpy as jnp
from jax.experimental import pallas as pl


def kernel(rpn_out, rois, gt_anchor_label, gt_bbox, gt_label, W1, b1, W2, b2, W_loc, b_loc, W_cls, b_cls, img_h, img_w):
    raise NotImplementedError("write your pallas kernel here")


# speedup vs baseline: 3.1496x; 3.1496x over previous
import jax
import jax.numpy as jnp
from jax.experimental import pallas as pl
from jax.experimental.pallas import tpu as pltpu

N_ROIS = 2000
NPAD = 2048
NUM_CLASSES = 81
IN_CH = 25088
N_GT = 64
IOU_BG_THRESH = 0.4
NMS_THRESH = 0.5

BLK_M = 256
BLK_K = 1792
KB = IN_CH // BLK_K
MB = NPAD // BLK_M
LOC_PAD = 384
CLS_PAD = 128


def _mlp_kernel(x_ref, w1_ref, b1_ref, w2_ref, b2_ref, wloc_ref, bloc_ref,
                wcls_ref, bcls_ref, lab_ref, cls_ref, misc_ref, acc_ref):
    k = pl.program_id(1)

    @pl.when(k == 0)
    def _():
        acc_ref[...] = jnp.zeros_like(acc_ref)

    bf = jnp.bfloat16
    acc_ref[...] += jnp.dot(x_ref[...].astype(bf), w1_ref[...].astype(bf),
                            preferred_element_type=jnp.float32)

    @pl.when(k == KB - 1)
    def _():
        h1 = jnp.maximum(acc_ref[...] + b1_ref[...], 0.0)
        h2 = jnp.maximum(
            jnp.dot(h1.astype(bf), w2_ref[...].astype(bf),
                    preferred_element_type=jnp.float32)
            + b2_ref[...], 0.0)
        logits = (jnp.dot(h2.astype(bf), wcls_ref[...].astype(bf),
                          preferred_element_type=jnp.float32)
                  + bcls_ref[...])
        m = jnp.max(logits, axis=1, keepdims=True)
        e = jnp.exp(logits - m)
        z = jnp.sum(e, axis=1, keepdims=True)
        cls_ref[...] = e / z
        score = 1.0 / z
        loc_all = (jnp.dot(h2.astype(bf), wloc_ref[...].astype(bf),
                           preferred_element_type=jnp.float32)
                   + bloc_ref[...])
        lab = lab_ref[:, 0:1]
        col = jax.lax.broadcasted_iota(jnp.int32, (BLK_M, LOC_PAD), 1)
        col128 = jax.lax.broadcasted_iota(jnp.int32, (BLK_M, CLS_PAD), 1)
        misc = jnp.zeros((BLK_M, CLS_PAD), jnp.float32)
        for j in range(4):
            lj = jnp.sum(jnp.where(col == 4 * lab + j, loc_all, 0.0),
                         axis=1, keepdims=True)
            misc = misc + jnp.where(col128 == j, lj, 0.0)
        misc = misc + jnp.where(col128 == 4, score, 0.0)
        misc_ref[...] = misc


def _head_pallas(xp, w1, b1, w2, b2, wloc, bloc, wcls, bcls, labb):
    return pl.pallas_call(
        _mlp_kernel,
        grid=(MB, KB),
        in_specs=[
            pl.BlockSpec((BLK_M, BLK_K), lambda m, k: (m, k)),
            pl.BlockSpec((BLK_K, 1024), lambda m, k: (k, 0)),
            pl.BlockSpec((1, 1024), lambda m, k: (0, 0)),
            pl.BlockSpec((1024, 1024), lambda m, k: (0, 0)),
            pl.BlockSpec((1, 1024), lambda m, k: (0, 0)),
            pl.BlockSpec((1024, LOC_PAD), lambda m, k: (0, 0)),
            pl.BlockSpec((1, LOC_PAD), lambda m, k: (0, 0)),
            pl.BlockSpec((1024, CLS_PAD), lambda m, k: (0, 0)),
            pl.BlockSpec((1, CLS_PAD), lambda m, k: (0, 0)),
            pl.BlockSpec((BLK_M, 128), lambda m, k: (m, 0)),
        ],
        out_specs=[
            pl.BlockSpec((BLK_M, CLS_PAD), lambda m, k: (m, 0)),
            pl.BlockSpec((BLK_M, CLS_PAD), lambda m, k: (m, 0)),
        ],
        out_shape=[
            jax.ShapeDtypeStruct((NPAD, CLS_PAD), jnp.float32),
            jax.ShapeDtypeStruct((NPAD, CLS_PAD), jnp.float32),
        ],
        scratch_shapes=[pltpu.VMEM((BLK_M, 1024), jnp.float32)],
        compiler_params=pltpu.CompilerParams(
            dimension_semantics=("parallel", "arbitrary")),
    )(xp, w1, b1, w2, b2, wloc, bloc, wcls, bcls, labb)


def _geo_kernel(ly_ref, lx_ref, lh_ref, lw_ref, sc_ref,
                rx1_ref, ry1_ref, rx2_ref, ry2_ref,
                gy_ref, gx_ref, gh_ref, gw_ref, glab_ref,
                oly_ref, olx_ref, olh_ref, olw_ref, okeep_ref,
                ogy_ref, ogx_ref, ogh_ref, ogw_ref, oglab_ref,
                opy_ref, opx_ref, oph_ref, opw_ref, osc_ref):
    shp = (16, 128)
    idx = (jax.lax.broadcasted_iota(jnp.int32, shp, 0) * 128
           + jax.lax.broadcasted_iota(jnp.int32, shp, 1))
    valid = idx < N_ROIS
    validf = valid.astype(jnp.float32)

    ly, lx, lh, lw = ly_ref[...], lx_ref[...], lh_ref[...], lw_ref[...]
    score = sc_ref[...]

    cnt = 4.0 * N_ROIS
    tot = (jnp.sum(ly * validf) + jnp.sum(lx * validf)
           + jnp.sum(lh * validf) + jnp.sum(lw * validf))
    mean = tot / cnt
    sq = (jnp.sum((ly - mean) ** 2 * validf) + jnp.sum((lx - mean) ** 2 * validf)
          + jnp.sum((lh - mean) ** 2 * validf) + jnp.sum((lw - mean) ** 2 * validf))
    std = jnp.sqrt(sq / (cnt - 1.0))
    ly, lx, lh, lw = ((ly - mean) / std, (lx - mean) / std,
                      (lh - mean) / std, (lw - mean) / std)

    ry = ry1_ref[...]
    rx = rx1_ref[...]
    rh = ry2_ref[...] - ry1_ref[...]
    rw = rx2_ref[...] - rx1_ref[...]

    cy = ry + 0.5 * rh
    cx = rx + 0.5 * rw
    pcy = ly * rh + cy
    pcx = lx * rw + cx
    ph = rh * jnp.exp(lh)
    pw = rw * jnp.exp(lw)
    py = pcy - 0.5 * ph
    px = pcx - 0.5 * pw

    ay2, ax2 = py + ph, px + pw
    area_a = ph * pw
    iou_max = jnp.full(shp, -1.0, jnp.float32)
    bbox_idx = jnp.zeros(shp, jnp.int32)
    for g in range(N_GT):
        gyv, gxv = gy_ref[0, g], gx_ref[0, g]
        ghv, gwv = gh_ref[0, g], gw_ref[0, g]
        ih2 = jnp.maximum(jnp.minimum(ay2, gyv + ghv) - jnp.maximum(py, gyv), 0.0)
        iw2 = jnp.maximum(jnp.minimum(ax2, gxv + gwv) - jnp.maximum(px, gxv), 0.0)
        inter = ih2 * iw2
        iou_g = inter / (area_a + ghv * gwv - inter)
        upd = iou_g > iou_max
        bbox_idx = jnp.where(upd, g, bbox_idx)
        iou_max = jnp.maximum(iou_max, iou_g)
    gsel_y = jnp.zeros(shp, jnp.float32)
    gsel_x = jnp.zeros(shp, jnp.float32)
    gsel_h = jnp.zeros(shp, jnp.float32)
    gsel_w = jnp.zeros(shp, jnp.float32)
    gsel_lab = jnp.zeros(shp, jnp.int32)
    for g in range(N_GT):
        hit = bbox_idx == g
        gsel_y = jnp.where(hit, gy_ref[0, g], gsel_y)
        gsel_x = jnp.where(hit, gx_ref[0, g], gsel_x)
        gsel_h = jnp.where(hit, gh_ref[0, g], gsel_h)
        gsel_w = jnp.where(hit, gw_ref[0, g], gsel_w)
        gsel_lab = jnp.where(hit, glab_ref[0, g], gsel_lab)
    groi_lab = jnp.where(iou_max < IOU_BG_THRESH, 0, gsel_lab)

    gcy = gsel_y + 0.5 * gsel_h
    gcx = gsel_x + 0.5 * gsel_w
    gdy = (gcy - cy) / rh
    gdx = (gcx - cx) / rw
    gdh = jnp.log(gsel_h / rh)
    gdw = jnp.log(gsel_w / rw)

    nx1, ny1, nx2, ny2 = px, py, px + pw, py + ph
    areas = pw * ph
    sc_safe = jnp.where(valid, score, -1.0)

    def cond(state):
        alive_f, keep_f, t = state
        return jnp.logical_and(t < NPAD, jnp.max(alive_f * (1.0 - keep_f)) > 0.5)

    def body(state):
        alive_f, keep_f, t = state
        cand = alive_f * (1.0 - keep_f) > 0.5
        s_c = jnp.where(cand, sc_safe, -2.0)
        mx = jnp.max(s_c)
        i_sel = jnp.min(jnp.where(jnp.logical_and(cand, s_c == mx), idx, NPAD))
        sel = idx == i_sel
        bx1 = jnp.sum(jnp.where(sel, nx1, 0.0))
        by1 = jnp.sum(jnp.where(sel, ny1, 0.0))
        bx2 = jnp.sum(jnp.where(sel, nx2, 0.0))
        by2 = jnp.sum(jnp.where(sel, ny2, 0.0))
        bar = jnp.sum(jnp.where(sel, areas, 0.0))
        iw = jnp.maximum(jnp.minimum(bx2, nx2) - jnp.maximum(bx1, nx1), 0.0)
        ih = jnp.maximum(jnp.minimum(by2, ny2) - jnp.maximum(by1, ny1), 0.0)
        inter2 = iw * ih
        iou = inter2 / (bar + areas - inter2)
        sup = jnp.logical_and(
            jnp.logical_and(iou > NMS_THRESH, alive_f > 0.5), ~sel)
        alive_f = jnp.where(sup, 0.0, alive_f)
        keep_f = jnp.where(sel, 1.0, keep_f)
        return (alive_f, keep_f, t + 1)

    alive0 = validf
    keep0 = jnp.zeros(shp, jnp.float32)
    _, mf, _ = jax.lax.while_loop(cond, body, (alive0, keep0, jnp.int32(0)))
    oly_ref[...] = ly * mf
    olx_ref[...] = lx * mf
    olh_ref[...] = lh * mf
    olw_ref[...] = lw * mf
    okeep_ref[...] = mf
    ogy_ref[...] = gdy * mf
    ogx_ref[...] = gdx * mf
    ogh_ref[...] = gdh * mf
    ogw_ref[...] = gdw * mf
    oglab_ref[...] = groi_lab * mf.astype(jnp.int32)
    opy_ref[...] = py * mf
    opx_ref[...] = px * mf
    oph_ref[...] = ph * mf
    opw_ref[...] = pw * mf
    osc_ref[...] = score * mf


def _geo_pallas(*args):
    v = jax.ShapeDtypeStruct((16, 128), jnp.float32)
    vi = jax.ShapeDtypeStruct((16, 128), jnp.int32)
    return pl.pallas_call(
        _geo_kernel,
        out_shape=[v, v, v, v, v, v, v, v, v, vi, v, v, v, v, v],
    )(*args)


def kernel(rpn_out, rois, gt_anchor_label, gt_bbox, gt_label,
           W1, b1, W2, b2, W_loc, b_loc, W_cls, b_cls, img_h, img_w):
    f32 = jnp.float32
    xp = jnp.pad(rpn_out.astype(f32), ((0, NPAD - N_ROIS), (0, 0)))
    labp = jnp.pad(gt_anchor_label.astype(jnp.int32), (0, NPAD - N_ROIS))
    labb = jnp.broadcast_to(labp[:, None], (NPAD, 128))
    wloc = jnp.pad(W_loc.astype(f32), ((0, 0), (0, LOC_PAD - 4 * NUM_CLASSES)))
    bloc = jnp.pad(b_loc.astype(f32), (0, LOC_PAD - 4 * NUM_CLASSES))[None, :]
    wcls = jnp.pad(W_cls.astype(f32), ((0, 0), (0, CLS_PAD - NUM_CLASSES)))
    bcls = jnp.concatenate(
        [b_cls.astype(f32),
         jnp.full((CLS_PAD - NUM_CLASSES,), -1e30, f32)])[None, :]

    cls_full, misc = _head_pallas(
        xp, W1.astype(f32), b1.astype(f32)[None, :], W2.astype(f32),
        b2.astype(f32)[None, :], wloc, bloc, wcls, bcls, labb)

    def col16(a2d, j):
        return a2d[:, j].reshape(16, 128)

    def padcol16(a1d):
        return jnp.pad(a1d.astype(f32), (0, NPAD - N_ROIS)).reshape(16, 128)

    ins = (
        col16(misc, 0), col16(misc, 1), col16(misc, 2), col16(misc, 3),
        col16(misc, 4),
        padcol16(rois[:, 0]), padcol16(rois[:, 1]),
        padcol16(rois[:, 2]), padcol16(rois[:, 3]),
        gt_bbox[:, 0].astype(f32)[None, :], gt_bbox[:, 1].astype(f32)[None, :],
        gt_bbox[:, 2].astype(f32)[None, :], gt_bbox[:, 3].astype(f32)[None, :],
        gt_label.astype(jnp.int32)[None, :],
    )
    (oly, olx, olh, olw, okeep, ogy, ogx, ogh, ogw, oglab,
     opy, opx, oph, opw, osc) = _geo_pallas(*ins)

    def unflat(a):
        return a.reshape(-1)[:N_ROIS]

    loc = jnp.stack([unflat(oly), unflat(olx), unflat(olh), unflat(olw)], axis=1)
    cls_out = cls_full[:N_ROIS, :NUM_CLASSES] * unflat(okeep)[:, None]
    gt_roi_loc = jnp.stack([unflat(ogy), unflat(ogx), unflat(ogh), unflat(ogw)],
                           axis=1)
    gt_roi_label = unflat(oglab)
    pred_box = jnp.stack([unflat(opy), unflat(opx), unflat(oph), unflat(opw)],
                         axis=1)
    pred_box_score = unflat(osc)
    return (loc, cls_out, gt_roi_loc, gt_roi_label, pred_box, pred_box_score)
